# baseline (speedup 1.0000x reference)
import functools

import jax
import jax.numpy as jnp
from jax import lax
from jax.experimental import pallas as pl
from jax.experimental.pallas import tpu as pltpu
from jax.experimental.pallas import tpu_sc as plsc

_L = 16
_G = 128


def _sc_info():
    try:
        info = plsc.get_sparse_core_info()
        return info.num_cores, info.num_subcores
    except Exception:
        return 2, 16


def _compact_body(nw, rpt, nb, blk, cpad, ei_hbm, srcc_hbm, ldst_hbm,
                  cnts_hbm, dbuf, sbuf, cs, cl, cnt_v):
    nc, _ = _sc_info()
    wid = lax.axis_index("s") * nc + lax.axis_index("c")
    lo = wid * rpt
    hi = lo + rpt
    nch = blk // _L

    def zero_chunk(i, _):
        cs[pl.ds(i * _L, _L)] = jnp.zeros((_L,), jnp.int32)
        return 0

    lax.fori_loop(0, nch, zero_chunk, 0)

    def block(b, _):
        pltpu.sync_copy(ei_hbm.at[0, pl.ds(b * blk, blk)], sbuf)
        pltpu.sync_copy(ei_hbm.at[1, pl.ds(b * blk, blk)], dbuf)

        def chunk(i, off):
            d16 = dbuf[pl.ds(i * _L, _L)]
            s16 = sbuf[pl.ds(i * _L, _L)]
            m = (d16 >= lo) & (d16 < hi)
            mi = jnp.where(m, 1, 0).astype(jnp.int32)
            incl = plsc.cumsum(mi)
            idx = jnp.where(m, off + incl - mi, blk)
            plsc.store_scatter(cs, [idx], s16)
            plsc.store_scatter(cl, [idx], d16 - lo)
            return off + incl[_L - 1]

        cnt = lax.fori_loop(0, nch, chunk, 0)
        lane = lax.iota(jnp.int32, _L)
        cidx = jnp.where(lane == 0, b, cpad - 1)
        plsc.store_scatter(cnt_v, [cidx], jnp.full((_L,), cnt, jnp.int32))
        pltpu.sync_copy(cs.at[pl.ds(0, blk)], srcc_hbm.at[wid, b])
        pltpu.sync_copy(cl.at[pl.ds(0, blk)], ldst_hbm.at[wid, b])
        return 0

    lax.fori_loop(0, nb, block, 0)
    pltpu.sync_copy(cnt_v, cnts_hbm.at[wid])


def _segmax_body(nw, rpt, nb, blk, h_hbm, srcc_hbm, ldst_hbm, cnts_hbm,
                 out_hbm, cnt_v, sbuf, lbuf, rows, acc, sem):
    nc, _ = _sc_info()
    wid = lax.axis_index("s") * nc + lax.axis_index("c")
    lo = wid * rpt
    nvec = rpt * 128 // _L
    neg = jnp.full((_L,), -jnp.inf, jnp.float32)

    pltpu.sync_copy(cnts_hbm.at[wid], cnt_v)

    def init_chunk(i, _):
        acc[pl.ds(i * _L, _L)] = neg
        return 0

    lax.fori_loop(0, nvec, init_chunk, 0)

    def block(b, _):
        c = cnt_v[pl.ds(b, _L)][0]

        @pl.when(c > 0)
        def _():
            pltpu.sync_copy(srcc_hbm.at[wid, b], sbuf)
            pltpu.sync_copy(ldst_hbm.at[wid, b], lbuf.at[pl.ds(0, blk)])
            ngather = (c + _G - 1) // _G

            def gchunk(k, _):
                pltpu.async_copy(h_hbm.at[sbuf.at[pl.ds(k * _G, _G)]],
                                 rows, sem).wait()
                ne = jnp.minimum(_G, c - k * _G)

                def edge(e, _):
                    d = lbuf[pl.ds(k * _G + e, _L)][0]
                    base = d * 128
                    for j in range(8):
                        a = acc[pl.ds(base + j * _L, _L)]
                        v = rows[e, pl.ds(j * _L, _L)]
                        acc[pl.ds(base + j * _L, _L)] = jnp.maximum(a, v)
                    return 0

                lax.fori_loop(0, ne, edge, 0)
                return 0

            lax.fori_loop(0, ngather, gchunk, 0)

        return 0

    lax.fori_loop(0, nb, block, 0)

    def fix_chunk(i, _):
        a = acc[pl.ds(i * _L, _L)]
        acc[pl.ds(i * _L, _L)] = jnp.where(a == neg, 0.0, a)
        return 0

    lax.fori_loop(0, nvec, fix_chunk, 0)
    pltpu.sync_copy(acc, out_hbm.at[pl.ds(lo * 128, rpt * 128)])


def _sage_lin_tc(agg, h, wlT, bl, wrT):
    n = agg.shape[0]

    def body(a_ref, h_ref, wl_ref, bl_ref, wr_ref, o_ref):
        o_ref[...] = (
            jnp.dot(a_ref[...], wl_ref[...], preferred_element_type=jnp.float32)
            + jnp.dot(h_ref[...], wr_ref[...], preferred_element_type=jnp.float32)
            + bl_ref[...])

    return pl.pallas_call(
        body, out_shape=jax.ShapeDtypeStruct((n, wlT.shape[1]), jnp.float32),
    )(agg, h, wlT, bl.reshape(1, -1), wrT)


def _final_tc(agg, h, wlT, bl, wrT, w1T, b1, w2T, b2, w3T, b3):
    n = agg.shape[0]

    def body(a_ref, h_ref, wl_ref, bl_ref, wr_ref, w1_ref, b1_ref, w2_ref,
             b2_ref, w3_ref, b3_ref, o_ref):
        h3 = (jnp.dot(a_ref[...], wl_ref[...], preferred_element_type=jnp.float32)
              + jnp.dot(h_ref[...], wr_ref[...], preferred_element_type=jnp.float32)
              + bl_ref[...])
        t = jnp.maximum(
            jnp.dot(h3, w1_ref[...], preferred_element_type=jnp.float32)
            + b1_ref[...], 0.0)
        t = jnp.maximum(
            jnp.dot(t, w2_ref[...], preferred_element_type=jnp.float32)
            + b2_ref[...], 0.0)
        t = (jnp.dot(t, w3_ref[...], preferred_element_type=jnp.float32)
             + b3_ref[...])
        o_ref[...] = 1.0 / (1.0 + jnp.exp(-t))

    return pl.pallas_call(
        body, out_shape=jax.ShapeDtypeStruct((n, w3T.shape[1]), jnp.float32),
    )(agg, h, wlT, bl.reshape(1, -1), wrT, w1T, b1.reshape(1, -1), w2T,
      b2.reshape(1, -1), w3T, b3.reshape(1, -1))


def kernel(x, edge_index, batch, W1l, b1l, W1r, W2l, b2l, W2r, W3l, b3l, W3r,
           l1W, l1b, l2W, l2b, l3W, l3b):
    n, d = x.shape
    e = edge_index.shape[1]
    nc, ns = _sc_info()
    nw = nc * ns
    rpt = -(-n // nw)
    rpt = -(-rpt // 8) * 8
    n2 = nw * rpt

    blk = 6400
    while e % blk:
        blk //= 2
    nb = e // blk
    cpad = -(-nb // _L) * _L + _L

    mesh = plsc.VectorSubcoreMesh(core_axis_name="c", subcore_axis_name="s")

    compact = pl.kernel(
        functools.partial(_compact_body, nw, rpt, nb, blk, cpad),
        out_type=(
            jax.ShapeDtypeStruct((nw, nb, blk), jnp.int32),
            jax.ShapeDtypeStruct((nw, nb, blk), jnp.int32),
            jax.ShapeDtypeStruct((nw, cpad), jnp.int32),
        ),
        mesh=mesh,
        scratch_types=[
            pltpu.VMEM((blk,), jnp.int32),
            pltpu.VMEM((blk,), jnp.int32),
            pltpu.VMEM((blk + _L,), jnp.int32),
            pltpu.VMEM((blk + _L,), jnp.int32),
            pltpu.VMEM((cpad,), jnp.int32),
        ],
        compiler_params=pltpu.CompilerParams(needs_layout_passes=False),
    )

    segmax = pl.kernel(
        functools.partial(_segmax_body, nw, rpt, nb, blk),
        out_type=jax.ShapeDtypeStruct((n2 * 128,), jnp.float32),
        mesh=mesh,
        scratch_types=[
            pltpu.VMEM((cpad,), jnp.int32),
            pltpu.VMEM((blk,), jnp.int32),
            pltpu.VMEM((blk + _L,), jnp.int32),
            pltpu.VMEM((_G, 128), jnp.float32),
            pltpu.VMEM((rpt * 128,), jnp.float32),
            pltpu.SemaphoreType.DMA,
        ],
        compiler_params=pltpu.CompilerParams(needs_layout_passes=False),
    )

    src_c, ldst_c, cnts = compact(edge_index)

    def agg_of(hcur):
        flat = segmax(hcur, src_c, ldst_c, cnts)
        return flat.reshape(n2, 128)[:n]

    w3p = jnp.zeros((8, l3W.shape[1]), jnp.float32).at[:l3W.shape[0]].set(l3W)
    b3p = jnp.zeros((8,), jnp.float32).at[:l3b.shape[0]].set(l3b)

    h1 = _sage_lin_tc(agg_of(x), x, W1l.T, b1l, W1r.T)
    h2 = _sage_lin_tc(agg_of(h1), h1, W2l.T, b2l, W2r.T)
    out = _final_tc(agg_of(h2), h2, W3l.T, b3l, W3r.T, l1W.T, l1b, l2W.T,
                    l2b, w3p.T, b3p)
    return out[:, :l3W.shape[0]]


# speedup vs baseline: 1.0652x; 1.0652x over previous
import functools

import jax
import jax.numpy as jnp
from jax import lax
from jax.experimental import pallas as pl
from jax.experimental.pallas import tpu as pltpu
from jax.experimental.pallas import tpu_sc as plsc

_L = 16
_G = 128


def _sc_info():
    try:
        info = plsc.get_sparse_core_info()
        return info.num_cores, info.num_subcores
    except Exception:
        return 2, 16


def _compact_body(nw, rpt, nb, blk, cpad, ei_hbm, srcc_hbm, ldst_hbm,
                  cnts_hbm, dbuf, sbuf, cs, cl, cnt_v):
    nc, _ = _sc_info()
    wid = lax.axis_index("s") * nc + lax.axis_index("c")
    lo = wid * rpt
    hi = lo + rpt
    nch = blk // _L

    def zero_chunk(i, _):
        cs[pl.ds(i * _L, _L)] = jnp.zeros((_L,), jnp.int32)
        return 0

    lax.fori_loop(0, nch, zero_chunk, 0)

    def block(b, _):
        pltpu.sync_copy(ei_hbm.at[0, pl.ds(b * blk, blk)], sbuf)
        pltpu.sync_copy(ei_hbm.at[1, pl.ds(b * blk, blk)], dbuf)

        def chunk(i, off):
            d16 = dbuf[pl.ds(i * _L, _L)]
            s16 = sbuf[pl.ds(i * _L, _L)]
            m = (d16 >= lo) & (d16 < hi)
            mi = jnp.where(m, 1, 0).astype(jnp.int32)
            incl = plsc.cumsum(mi)
            idx = jnp.where(m, off + incl - mi, blk)
            plsc.store_scatter(cs, [idx], s16)
            plsc.store_scatter(cl, [idx], d16 - lo)
            return off + incl[_L - 1]

        cnt = lax.fori_loop(0, nch, chunk, 0)
        lane = lax.iota(jnp.int32, _L)
        cidx = jnp.where(lane == 0, b, cpad - 1)
        plsc.store_scatter(cnt_v, [cidx], jnp.full((_L,), cnt, jnp.int32))
        pltpu.sync_copy(cs.at[pl.ds(0, blk)], srcc_hbm.at[wid, b])
        pltpu.sync_copy(cl.at[pl.ds(0, blk)], ldst_hbm.at[wid, b])
        return 0

    lax.fori_loop(0, nb, block, 0)
    pltpu.sync_copy(cnt_v, cnts_hbm.at[wid])


def _segmax_body(nw, rpt, nb, blk, h_hbm, srcc_hbm, ldst_hbm, cnts_hbm,
                 out_hbm, cnt_v, sbuf, lbuf, rows, acc, sem):
    nc, _ = _sc_info()
    wid = lax.axis_index("s") * nc + lax.axis_index("c")
    lo = wid * rpt
    nvec = rpt * 128 // _L
    neg = jnp.full((_L,), -jnp.inf, jnp.float32)

    pltpu.sync_copy(cnts_hbm.at[wid], cnt_v)

    def init_chunk(i, _):
        acc[pl.ds(i * _L, _L)] = neg
        return 0

    lax.fori_loop(0, nvec, init_chunk, 0)

    def block(b, _):
        c = cnt_v[pl.ds(b, _L)][0]

        @pl.when(c > 0)
        def _():
            pltpu.sync_copy(srcc_hbm.at[wid, b], sbuf)
            pltpu.sync_copy(ldst_hbm.at[wid, b], lbuf.at[pl.ds(0, blk)])
            ngather = (c + _G - 1) // _G

            def upd(e, d):
                base = d * 128
                for j in range(8):
                    a = acc[pl.ds(base + j * _L, _L)]
                    v = rows[e, pl.ds(j * _L, _L)]
                    acc[pl.ds(base + j * _L, _L)] = jnp.maximum(a, v)

            def gchunk(k, _):
                pltpu.async_copy(h_hbm.at[sbuf.at[pl.ds(k * _G, _G)]],
                                 rows, sem).wait()
                ne = jnp.minimum(_G, c - k * _G)

                def group16(g, _):
                    dvec = lbuf[pl.ds(k * _G + g * _L, _L)]
                    for e2 in range(_L):
                        upd(g * _L + e2, dvec[e2])
                    return 0

                def edge(e, _):
                    upd(e, lbuf[pl.ds(k * _G + e, _L)][0])
                    return 0

                nfull = ne // _L
                lax.fori_loop(0, nfull, group16, 0)
                lax.fori_loop(nfull * _L, ne, edge, 0)
                return 0

            lax.fori_loop(0, ngather, gchunk, 0)

        return 0

    lax.fori_loop(0, nb, block, 0)

    def fix_chunk(i, _):
        a = acc[pl.ds(i * _L, _L)]
        acc[pl.ds(i * _L, _L)] = jnp.where(a == neg, 0.0, a)
        return 0

    lax.fori_loop(0, nvec, fix_chunk, 0)
    pltpu.sync_copy(acc, out_hbm.at[pl.ds(lo * 128, rpt * 128)])


def _sage_lin_tc(agg, h, wlT, bl, wrT):
    n = agg.shape[0]

    def body(a_ref, h_ref, wl_ref, bl_ref, wr_ref, o_ref):
        o_ref[...] = (
            jnp.dot(a_ref[...], wl_ref[...], preferred_element_type=jnp.float32)
            + jnp.dot(h_ref[...], wr_ref[...], preferred_element_type=jnp.float32)
            + bl_ref[...])

    return pl.pallas_call(
        body, out_shape=jax.ShapeDtypeStruct((n, wlT.shape[1]), jnp.float32),
    )(agg, h, wlT, bl.reshape(1, -1), wrT)


def _final_tc(agg, h, wlT, bl, wrT, w1T, b1, w2T, b2, w3T, b3):
    n = agg.shape[0]

    def body(a_ref, h_ref, wl_ref, bl_ref, wr_ref, w1_ref, b1_ref, w2_ref,
             b2_ref, w3_ref, b3_ref, o_ref):
        h3 = (jnp.dot(a_ref[...], wl_ref[...], preferred_element_type=jnp.float32)
              + jnp.dot(h_ref[...], wr_ref[...], preferred_element_type=jnp.float32)
              + bl_ref[...])
        t = jnp.maximum(
            jnp.dot(h3, w1_ref[...], preferred_element_type=jnp.float32)
            + b1_ref[...], 0.0)
        t = jnp.maximum(
            jnp.dot(t, w2_ref[...], preferred_element_type=jnp.float32)
            + b2_ref[...], 0.0)
        t = (jnp.dot(t, w3_ref[...], preferred_element_type=jnp.float32)
             + b3_ref[...])
        o_ref[...] = 1.0 / (1.0 + jnp.exp(-t))

    return pl.pallas_call(
        body, out_shape=jax.ShapeDtypeStruct((n, w3T.shape[1]), jnp.float32),
    )(agg, h, wlT, bl.reshape(1, -1), wrT, w1T, b1.reshape(1, -1), w2T,
      b2.reshape(1, -1), w3T, b3.reshape(1, -1))


def kernel(x, edge_index, batch, W1l, b1l, W1r, W2l, b2l, W2r, W3l, b3l, W3r,
           l1W, l1b, l2W, l2b, l3W, l3b):
    n, d = x.shape
    e = edge_index.shape[1]
    nc, ns = _sc_info()
    nw = nc * ns
    rpt = -(-n // nw)
    rpt = -(-rpt // 8) * 8
    n2 = nw * rpt

    blk = 6400
    while e % blk:
        blk //= 2
    nb = e // blk
    cpad = -(-nb // _L) * _L + _L

    mesh = plsc.VectorSubcoreMesh(core_axis_name="c", subcore_axis_name="s")

    compact = pl.kernel(
        functools.partial(_compact_body, nw, rpt, nb, blk, cpad),
        out_type=(
            jax.ShapeDtypeStruct((nw, nb, blk), jnp.int32),
            jax.ShapeDtypeStruct((nw, nb, blk), jnp.int32),
            jax.ShapeDtypeStruct((nw, cpad), jnp.int32),
        ),
        mesh=mesh,
        scratch_types=[
            pltpu.VMEM((blk,), jnp.int32),
            pltpu.VMEM((blk,), jnp.int32),
            pltpu.VMEM((blk + _L,), jnp.int32),
            pltpu.VMEM((blk + _L,), jnp.int32),
            pltpu.VMEM((cpad,), jnp.int32),
        ],
        compiler_params=pltpu.CompilerParams(needs_layout_passes=False),
    )

    segmax = pl.kernel(
        functools.partial(_segmax_body, nw, rpt, nb, blk),
        out_type=jax.ShapeDtypeStruct((n2 * 128,), jnp.float32),
        mesh=mesh,
        scratch_types=[
            pltpu.VMEM((cpad,), jnp.int32),
            pltpu.VMEM((blk,), jnp.int32),
            pltpu.VMEM((blk + _L,), jnp.int32),
            pltpu.VMEM((_G, 128), jnp.float32),
            pltpu.VMEM((rpt * 128,), jnp.float32),
            pltpu.SemaphoreType.DMA,
        ],
        compiler_params=pltpu.CompilerParams(needs_layout_passes=False),
    )

    src_c, ldst_c, cnts = compact(edge_index)

    def agg_of(hcur):
        flat = segmax(hcur, src_c, ldst_c, cnts)
        return flat.reshape(n2, 128)[:n]

    w3p = jnp.zeros((8, l3W.shape[1]), jnp.float32).at[:l3W.shape[0]].set(l3W)
    b3p = jnp.zeros((8,), jnp.float32).at[:l3b.shape[0]].set(l3b)

    h1 = _sage_lin_tc(agg_of(x), x, W1l.T, b1l, W1r.T)
    h2 = _sage_lin_tc(agg_of(h1), h1, W2l.T, b2l, W2r.T)
    out = _final_tc(agg_of(h2), h2, W3l.T, b3l, W3r.T, l1W.T, l1b, l2W.T,
                    l2b, w3p.T, b3p)
    return out[:, :l3W.shape[0]]
